# baseline (speedup 1.0000x reference)
import functools

import jax
import jax.numpy as jnp
from jax import lax
from jax.experimental import pallas as pl
from jax.experimental.pallas import tpu as pltpu
from jax.experimental.pallas import tpu_sc as plsc

_NC, _NS, _L = 2, 16, 16
_NW = _NC * _NS
_CHUNK = 128


def kernel(users, items, user_table, movie_table):
    B = users.shape[0]
    N, D = user_table.shape
    DW = 4 * D
    b_per_w = B // _NW
    n_chunks = b_per_w // _CHUNK
    ut4 = user_table.reshape(N // 4, DW)
    mt4 = movie_table.reshape(N // 4, DW)
    mesh = plsc.VectorSubcoreMesh(core_axis_name="c", subcore_axis_name="s")

    @functools.partial(
        pl.kernel,
        out_type=jax.ShapeDtypeStruct((B,), jnp.float32),
        mesh=mesh,
        scratch_types=[
            pltpu.VMEM((b_per_w,), jnp.int32),
            pltpu.VMEM((b_per_w,), jnp.int32),
            pltpu.VMEM((b_per_w,), jnp.int32),
            pltpu.VMEM((b_per_w,), jnp.int32),
            pltpu.VMEM((_CHUNK, DW), jnp.float32),
            pltpu.VMEM((_CHUNK, DW), jnp.float32),
            pltpu.VMEM((_CHUNK, DW), jnp.float32),
            pltpu.VMEM((_CHUNK, DW), jnp.float32),
            pltpu.VMEM((b_per_w,), jnp.float32),
            pltpu.SemaphoreType.DMA,
            pltpu.SemaphoreType.DMA,
        ],
        compiler_params=pltpu.CompilerParams(
            needs_layout_passes=False, use_tc_tiling_on_sc=False),
    )
    def mf_kernel(users_hbm, items_hbm, ut_hbm, mt_hbm, out_hbm,
                  uidx_v, iidx_v, uidx4_v, iidx4_v,
                  ubuf0, ubuf1, mbuf0, mbuf1, out_v, usem, msem):
        wid = lax.axis_index("s") * _NC + lax.axis_index("c")
        base = wid * b_per_w
        pltpu.sync_copy(users_hbm.at[pl.ds(base, b_per_w)], uidx_v)
        pltpu.sync_copy(items_hbm.at[pl.ds(base, b_per_w)], iidx_v)

        def phys_body(j, carry):
            sl = pl.ds(j * _L, _L)
            uidx4_v[sl] = lax.shift_right_logical(uidx_v[sl], 2)
            iidx4_v[sl] = lax.shift_right_logical(iidx_v[sl], 2)
            return carry

        lax.fori_loop(0, b_per_w // _L, phys_body, 0)

        ubufs = (ubuf0, ubuf1)
        mbufs = (mbuf0, mbuf1)

        def gather_chunk(c):
            sl = pl.ds(c * _CHUNK, _CHUNK)
            return (
                pltpu.async_copy(ut_hbm.at[uidx4_v.at[sl]], ubufs[c % 2], usem),
                pltpu.async_copy(mt_hbm.at[iidx4_v.at[sl]], mbufs[c % 2], msem),
            )

        pending = gather_chunk(0)
        for c in range(n_chunks):
            nxt = gather_chunk(c + 1) if c + 1 < n_chunks else None
            for cp in pending:
                cp.wait()
            pending = nxt
            ub, mb = ubufs[c % 2], mbufs[c % 2]

            def group_body(g, carry, ub=ub, mb=mb, c=c):
                k = g * _L + lax.iota(jnp.int32, _L)
                sl_b = pl.ds(c * _CHUNK + g * _L, _L)
                ucb = lax.shift_left(uidx_v[sl_b] & 3, 5)
                mcb = lax.shift_left(iidx_v[sl_b] & 3, 5)
                acc = jnp.zeros((_L,), jnp.float32)
                for d in range(D):
                    u = plsc.load_gather(ub, [k, ucb + d])
                    m = plsc.load_gather(mb, [k, mcb + d])
                    acc = acc + u * m
                out_v[sl_b] = 1.0 / (1.0 + jnp.exp(-acc))
                return carry

            lax.fori_loop(0, _CHUNK // _L, group_body, 0)

        pltpu.sync_copy(out_v, out_hbm.at[pl.ds(base, b_per_w)])

    return mf_kernel(users, items, ut4, mt4)
